# Initial kernel scaffold; baseline (speedup 1.0000x reference)
import jax
import jax.numpy as jnp
from jax.experimental import pallas as pl


def kernel(query, key, value, query_pos, reference_points, spatial_shapes, level_start_index, im2col_step, W_so, b_so, W_aw, b_aw, W_vp, b_vp, W_op, b_op):
    raise NotImplementedError("write your pallas kernel here")


# speedup vs baseline: 26.4552x; 26.4552x over previous
import functools
import numpy as np
import jax
import jax.numpy as jnp
from jax import lax
from jax.experimental import pallas as pl
from jax.experimental.pallas import tpu as pltpu
from jax.experimental.pallas import tpu_sc as plsc

NQ = 10000
EMB = 256
NH = 8
NP = 4
NBQ = 2
H0 = 100
W0 = 100
DH = EMB // NH
LPH = NBQ * NP * 4
EPQ = NH * LPH

NW = 32
NQ_PAD = 10240
QPW = NQ_PAD // NW
G = 4
CPW = QPW // G
K = G * EPQ
SPQ = NH * NBQ * NP
KR = G * SPQ
HP = H0 + 2
WP = W0 + 2

TQ = 1000


def _stage_a_body(val0_ref, q_ref, qp_ref, wsox_v_ref, wsox_q_ref,
                  wsoy_v_ref, wsoy_q_ref, waw_v_ref, waw_q_ref,
                  bsox_ref, bsoy_ref, baw_ref, smask_ref,
                  refx_ref, refy_ref, base_ref,
                  ib_ref, w00_ref, w01_ref, w10_ref, w11_ref):
    f32 = jnp.float32
    v0 = val0_ref[...]
    qq = q_ref[...] + qp_ref[...]
    sox = (jnp.dot(v0, wsox_v_ref[...], preferred_element_type=f32)
           + jnp.dot(qq, wsox_q_ref[...], preferred_element_type=f32)
           + bsox_ref[...])
    soy = (jnp.dot(v0, wsoy_v_ref[...], preferred_element_type=f32)
           + jnp.dot(qq, wsoy_q_ref[...], preferred_element_type=f32)
           + bsoy_ref[...])
    logits = (jnp.dot(v0, waw_v_ref[...], preferred_element_type=f32)
              + jnp.dot(qq, waw_q_ref[...], preferred_element_type=f32)
              + baw_ref[...])
    e = jnp.exp(logits)
    denom = jnp.dot(e, smask_ref[...], preferred_element_type=f32)
    aw = e / denom

    px = refx_ref[...] * W0 + sox - 0.5
    py = refy_ref[...] * H0 + soy - 0.5
    x0f = jnp.floor(px)
    y0f = jnp.floor(py)
    fx = px - x0f
    fy = py - y0f
    x1f = x0f + 1.0
    y1f = y0f + 1.0
    vx0 = (x0f >= 0.0) & (x0f <= W0 - 1.0)
    vx1 = (x1f >= 0.0) & (x1f <= W0 - 1.0)
    vy0 = (y0f >= 0.0) & (y0f <= H0 - 1.0)
    vy1 = (y1f >= 0.0) & (y1f <= H0 - 1.0)
    xbi = (jnp.clip(x0f, -1.0, float(W0)) + 1.0).astype(jnp.int32)
    ybi = (jnp.clip(y0f, -1.0, float(H0)) + 1.0).astype(jnp.int32)

    base = base_ref[...]
    ib_ref[...] = base + ybi * WP + xbi
    half_aw = 0.5 * aw
    wx0 = 1.0 - fx
    wy0 = 1.0 - fy
    z = jnp.zeros_like(aw)
    w00_ref[...] = jnp.where(vx0 & vy0, half_aw * wx0 * wy0, z)
    w01_ref[...] = jnp.where(vx1 & vy0, half_aw * fx * wy0, z)
    w10_ref[...] = jnp.where(vx0 & vy1, half_aw * wx0 * fy, z)
    w11_ref[...] = jnp.where(vx1 & vy1, half_aw * fx * fy, z)


def _vproj_body(val_ref, wvp_ref, bvp_ref, out_ref):
    vt = (jnp.dot(val_ref[0], wvp_ref[...], preferred_element_type=jnp.float32)
          + bvp_ref[...])
    for h in range(NH):
        out_ref[0, h] = vt[:, h * DH:(h + 1) * DH]


def _stage_c_body(x_ref, wop_ref, bop_ref, q_ref, out_ref):
    out_ref[...] = (jnp.dot(x_ref[...], wop_ref[...],
                            preferred_element_type=jnp.float32)
                    + bop_ref[...] + q_ref[...])


def _sc_gather_kernel(table_hbm, idx_hbm, wt_hbm, out_hbm,
                      idx_v, wt_v, rows_v, out_v, sem):
    wid = lax.axis_index("s") * 2 + lax.axis_index("c")

    def chunk_body(c, carry):
        irow = pl.multiple_of(wid * (CPW * 2) + c * 2, 2)
        bent = pl.multiple_of(wid * (QPW * EPQ) + c * K, 8)
        qb = pl.multiple_of(wid * QPW + c * G, 4)
        pltpu.sync_copy(idx_hbm.at[pl.ds(irow, 2)], idx_v)
        pltpu.sync_copy(wt_hbm.at[pl.ds(bent, K)], wt_v.at[pl.ds(0, K)])
        copies = [
            pltpu.async_copy(table_hbm.at[idx_v.at[i]],
                             rows_v.at[pl.ds(i * 128, 128)], sem)
            for i in range(2)
        ]
        for cp in copies:
            cp.wait()
        for g in range(G):
            for h in range(NH):
                bw = g * EPQ + h * (NBQ * NP * 4)
                br = g * SPQ + h * (NBQ * NP)

                def sbody(s, acc):
                    a0, a1 = acc
                    w4 = wt_v[pl.ds(bw + s * 4, 16)]
                    r = br + s
                    for c4 in range(4):
                        a0 = a0 + w4[c4] * rows_v[r, pl.ds(c4 * DH, 16)]
                        a1 = a1 + w4[c4] * rows_v[r, pl.ds(c4 * DH + 16, 16)]
                    return (a0, a1)

                acc0, acc1 = lax.fori_loop(
                    0, NBQ * NP, sbody,
                    (jnp.zeros((16,), jnp.float32),
                     jnp.zeros((16,), jnp.float32)))
                out_v[g, pl.ds(h * DH, 16)] = acc0
                out_v[g, pl.ds(h * DH + 16, 16)] = acc1
        pltpu.sync_copy(out_v, out_hbm.at[pl.ds(qb, G)])
        return carry

    lax.fori_loop(0, CPW, chunk_body, 0)


def kernel(query, key, value, query_pos, reference_points, spatial_shapes,
           level_start_index, im2col_step, W_so, b_so, W_aw, b_aw,
           W_vp, b_vp, W_op, b_op):
    f32 = jnp.float32
    q0 = query[0]
    qp0 = query_pos[0]
    val0 = value[0]

    wsox = W_so[0::2].T
    wsoy = W_so[1::2].T
    waw = W_aw.T
    wsox_v, wsox_q = wsox[:EMB], wsox[EMB:]
    wsoy_v, wsoy_q = wsoy[:EMB], wsoy[EMB:]
    waw_v, waw_q = waw[:EMB], waw[EMB:]
    bsox = b_so[0::2].reshape(1, 64)
    bsoy = b_so[1::2].reshape(1, 64)
    baw = b_aw.reshape(1, 64)

    lidx = np.arange(64)
    smask = jnp.asarray((lidx[:, None] // NP) == (lidx[None, :] // NP), f32)
    hh = lidx // (NBQ * NP)
    bb = (lidx // NP) % NBQ
    base64 = jnp.asarray(((bb * NH + hh) * (HP * WP))[None, :], jnp.int32)

    refx = reference_points[:, :, 0, 0]
    refy = reference_points[:, :, 0, 1]
    refx64 = jnp.tile(jnp.repeat(refx.T, NP, axis=1), (1, NH))
    refy64 = jnp.tile(jnp.repeat(refy.T, NP, axis=1), (1, NH))

    ntq = NQ // TQ
    row_spec = pl.BlockSpec((TQ, EMB), lambda i: (i, 0))
    lane_spec = pl.BlockSpec((TQ, 64), lambda i: (i, 0))
    full_spec = lambda s: pl.BlockSpec(s, lambda i: tuple(0 for _ in s))
    outs_a = pl.pallas_call(
        _stage_a_body,
        grid=(ntq,),
        in_specs=[row_spec, row_spec, row_spec,
                  full_spec((EMB, 64)), full_spec((EMB, 64)),
                  full_spec((EMB, 64)), full_spec((EMB, 64)),
                  full_spec((EMB, 64)), full_spec((EMB, 64)),
                  full_spec((1, 64)), full_spec((1, 64)), full_spec((1, 64)),
                  full_spec((64, 64)),
                  lane_spec, lane_spec, full_spec((1, 64))],
        out_specs=[lane_spec] * 5,
        out_shape=[jax.ShapeDtypeStruct((NQ, 64), jnp.int32)]
        + [jax.ShapeDtypeStruct((NQ, 64), f32)] * 4,
    )(val0, q0, qp0, wsox_v, wsox_q, wsoy_v, wsoy_q, waw_v, waw_q,
      bsox, bsoy, baw, smask, refx64, refy64, base64)
    ibase, w00, w01, w10, w11 = outs_a

    idx = jnp.pad(ibase, ((0, NQ_PAD - NQ), (0, 0))).reshape(-1, 128)
    wt = jnp.stack([w00, w01, w10, w11], axis=-1).reshape(NQ, EPQ)
    wt = jnp.pad(wt, ((0, NQ_PAD - NQ), (0, 0))).reshape(-1)

    table = pl.pallas_call(
        _vproj_body,
        grid=(NBQ, ntq),
        in_specs=[pl.BlockSpec((1, TQ, EMB), lambda b, i: (b, i, 0)),
                  pl.BlockSpec((EMB, EMB), lambda b, i: (0, 0)),
                  pl.BlockSpec((1, EMB), lambda b, i: (0, 0))],
        out_specs=pl.BlockSpec((1, NH, TQ, DH), lambda b, i: (b, 0, i, 0)),
        out_shape=jax.ShapeDtypeStruct((NBQ, NH, NQ, DH), f32),
    )(value, W_vp.T, b_vp.reshape(1, EMB))
    vg = table.reshape(NBQ, NH, H0, W0, DH)
    vp = jnp.pad(vg, ((0, 0), (0, 0), (1, 2), (1, 2), (0, 0)))
    t4 = jnp.stack([vp[:, :, 0:HP, 0:WP], vp[:, :, 0:HP, 1:WP + 1],
                    vp[:, :, 1:HP + 1, 0:WP], vp[:, :, 1:HP + 1, 1:WP + 1]],
                   axis=4)
    table = t4.reshape(NBQ * NH * HP * WP, 4 * DH)

    mesh = plsc.VectorSubcoreMesh(core_axis_name="c", subcore_axis_name="s")
    sc_fn = functools.partial(
        pl.kernel,
        mesh=mesh,
        out_type=jax.ShapeDtypeStruct((NQ_PAD, EMB), f32),
        scratch_types=[
            pltpu.VMEM((2, 128), jnp.int32),
            pltpu.VMEM((K + 16,), f32),
            pltpu.VMEM((KR, 4 * DH), f32),
            pltpu.VMEM((G, EMB), f32),
            pltpu.SemaphoreType.DMA,
        ],
    )(_sc_gather_kernel)
    msda_pad = sc_fn(table, idx, wt)

    msda = msda_pad[:NQ]
    out = pl.pallas_call(
        _stage_c_body,
        grid=(ntq,),
        in_specs=[row_spec, full_spec((EMB, EMB)), full_spec((1, EMB)),
                  row_spec],
        out_specs=row_spec,
        out_shape=jax.ShapeDtypeStruct((NQ, EMB), f32),
    )(msda, W_op.T, b_op.reshape(1, EMB), q0)
    return out.reshape(1, NQ, EMB)


# speedup vs baseline: 29.7246x; 1.1236x over previous
import functools
import numpy as np
import jax
import jax.numpy as jnp
from jax import lax
from jax.experimental import pallas as pl
from jax.experimental.pallas import tpu as pltpu
from jax.experimental.pallas import tpu_sc as plsc

NQ = 10000
EMB = 256
NH = 8
NP = 4
NBQ = 2
H0 = 100
W0 = 100
DH = EMB // NH
LPH = NBQ * NP * 4
EPQ = NH * LPH

NW = 32
NQ_PAD = 10240
QPW = NQ_PAD // NW
G = 4
CPW = QPW // G
K = G * EPQ
SPQ = NH * NBQ * NP
KR = G * SPQ
HP = H0 + 2
WP = W0 + 2

TQ = 1000


def _stage_a_body(val0_ref, q_ref, qp_ref, wsox_v_ref, wsox_q_ref,
                  wsoy_v_ref, wsoy_q_ref, waw_v_ref, waw_q_ref,
                  bsox_ref, bsoy_ref, baw_ref, smask_ref,
                  refx_ref, refy_ref, base_ref,
                  ib_ref, w00_ref, w01_ref, w10_ref, w11_ref):
    f32 = jnp.float32
    v0 = val0_ref[...]
    qq = q_ref[...] + qp_ref[...]
    sox = (jnp.dot(v0, wsox_v_ref[...], preferred_element_type=f32)
           + jnp.dot(qq, wsox_q_ref[...], preferred_element_type=f32)
           + bsox_ref[...])
    soy = (jnp.dot(v0, wsoy_v_ref[...], preferred_element_type=f32)
           + jnp.dot(qq, wsoy_q_ref[...], preferred_element_type=f32)
           + bsoy_ref[...])
    logits = (jnp.dot(v0, waw_v_ref[...], preferred_element_type=f32)
              + jnp.dot(qq, waw_q_ref[...], preferred_element_type=f32)
              + baw_ref[...])
    e = jnp.exp(logits)
    denom = jnp.dot(e, smask_ref[...], preferred_element_type=f32)
    aw = e / denom

    px = refx_ref[...] * W0 + sox - 0.5
    py = refy_ref[...] * H0 + soy - 0.5
    x0f = jnp.floor(px)
    y0f = jnp.floor(py)
    fx = px - x0f
    fy = py - y0f
    x1f = x0f + 1.0
    y1f = y0f + 1.0
    vx0 = (x0f >= 0.0) & (x0f <= W0 - 1.0)
    vx1 = (x1f >= 0.0) & (x1f <= W0 - 1.0)
    vy0 = (y0f >= 0.0) & (y0f <= H0 - 1.0)
    vy1 = (y1f >= 0.0) & (y1f <= H0 - 1.0)
    xbi = (jnp.clip(x0f, -1.0, float(W0)) + 1.0).astype(jnp.int32)
    ybi = (jnp.clip(y0f, -1.0, float(H0)) + 1.0).astype(jnp.int32)

    base = base_ref[...]
    ib_ref[...] = base + ybi * WP + xbi
    half_aw = 0.5 * aw
    wx0 = 1.0 - fx
    wy0 = 1.0 - fy
    z = jnp.zeros_like(aw)
    w00_ref[...] = jnp.where(vx0 & vy0, half_aw * wx0 * wy0, z)
    w01_ref[...] = jnp.where(vx1 & vy0, half_aw * fx * wy0, z)
    w10_ref[...] = jnp.where(vx0 & vy1, half_aw * wx0 * fy, z)
    w11_ref[...] = jnp.where(vx1 & vy1, half_aw * fx * fy, z)


def _vproj_body(val_ref, wvp_ref, bvp_ref, out_ref):
    vt = (jnp.dot(val_ref[0], wvp_ref[...], preferred_element_type=jnp.float32)
          + bvp_ref[...])
    for h in range(NH):
        out_ref[0, h] = vt[:, h * DH:(h + 1) * DH]


def _stage_c_body(x_ref, wop_ref, bop_ref, q_ref, out_ref):
    out_ref[...] = (jnp.dot(x_ref[...], wop_ref[...],
                            preferred_element_type=jnp.float32)
                    + bop_ref[...] + q_ref[...])


def _sc_gather_kernel(table_hbm, idx_hbm, wt_hbm, out_hbm,
                      idx_a, idx_b, wt_v, rows_a, rows_b, out_v,
                      sem_a, sem_b):
    wid = lax.axis_index("s") * 2 + lax.axis_index("c")
    ibase = wid * (CPW * 2)
    ebase = wid * (QPW * EPQ)
    qbase = wid * QPW

    def start_gathers(c, idx_v, rows_v, sem):
        pltpu.sync_copy(idx_hbm.at[pl.ds(ibase + c * 2, 2)], idx_v)
        for i in range(2):
            pltpu.async_copy(table_hbm.at[idx_v.at[i]],
                             rows_v.at[pl.ds(i * 128, 128)], sem)

    def drain_gathers(idx_v, rows_v, sem):
        for i in range(2):
            pltpu.make_async_copy(table_hbm.at[idx_v.at[i]],
                                  rows_v.at[pl.ds(i * 128, 128)], sem).wait()

    def compute_chunk(c, rows_v):
        pltpu.sync_copy(wt_hbm.at[pl.ds(ebase + c * K, K)],
                        wt_v.at[pl.ds(0, K)])
        for g in range(G):
            for h in range(NH):
                bw = g * EPQ + h * (NBQ * NP * 4)
                br = g * SPQ + h * (NBQ * NP)

                def sbody(s, acc):
                    a0, a1 = acc
                    w4 = wt_v[pl.ds(bw + s * 4, 16)]
                    r = br + s
                    for c4 in range(4):
                        a0 = a0 + w4[c4] * rows_v[r, pl.ds(c4 * DH, 16)]
                        a1 = a1 + w4[c4] * rows_v[r, pl.ds(c4 * DH + 16, 16)]
                    return (a0, a1)

                acc0, acc1 = lax.fori_loop(
                    0, NBQ * NP, sbody,
                    (jnp.zeros((16,), jnp.float32),
                     jnp.zeros((16,), jnp.float32)))
                out_v[g, pl.ds(h * DH, 16)] = acc0
                out_v[g, pl.ds(h * DH + 16, 16)] = acc1
        pltpu.sync_copy(out_v, out_hbm.at[pl.ds(qbase + c * G, G)])

    start_gathers(0, idx_a, rows_a, sem_a)

    def pair_body(t, carry):
        c0 = 2 * t
        start_gathers(c0 + 1, idx_b, rows_b, sem_b)
        drain_gathers(idx_a, rows_a, sem_a)
        compute_chunk(c0, rows_a)
        start_gathers(jnp.minimum(c0 + 2, CPW - 1), idx_a, rows_a, sem_a)
        drain_gathers(idx_b, rows_b, sem_b)
        compute_chunk(c0 + 1, rows_b)
        return carry

    lax.fori_loop(0, CPW // 2, pair_body, 0)
    drain_gathers(idx_a, rows_a, sem_a)


def kernel(query, key, value, query_pos, reference_points, spatial_shapes,
           level_start_index, im2col_step, W_so, b_so, W_aw, b_aw,
           W_vp, b_vp, W_op, b_op):
    f32 = jnp.float32
    q0 = query[0]
    qp0 = query_pos[0]
    val0 = value[0]

    wsox = W_so[0::2].T
    wsoy = W_so[1::2].T
    waw = W_aw.T
    wsox_v, wsox_q = wsox[:EMB], wsox[EMB:]
    wsoy_v, wsoy_q = wsoy[:EMB], wsoy[EMB:]
    waw_v, waw_q = waw[:EMB], waw[EMB:]
    bsox = b_so[0::2].reshape(1, 64)
    bsoy = b_so[1::2].reshape(1, 64)
    baw = b_aw.reshape(1, 64)

    lidx = np.arange(64)
    smask = jnp.asarray((lidx[:, None] // NP) == (lidx[None, :] // NP), f32)
    hh = lidx // (NBQ * NP)
    bb = (lidx // NP) % NBQ
    base64 = jnp.asarray(((bb * NH + hh) * (HP * WP))[None, :], jnp.int32)

    refx = reference_points[:, :, 0, 0]
    refy = reference_points[:, :, 0, 1]
    refx64 = jnp.tile(jnp.repeat(refx.T, NP, axis=1), (1, NH))
    refy64 = jnp.tile(jnp.repeat(refy.T, NP, axis=1), (1, NH))

    ntq = NQ // TQ
    row_spec = pl.BlockSpec((TQ, EMB), lambda i: (i, 0))
    lane_spec = pl.BlockSpec((TQ, 64), lambda i: (i, 0))
    full_spec = lambda s: pl.BlockSpec(s, lambda i: tuple(0 for _ in s))
    outs_a = pl.pallas_call(
        _stage_a_body,
        grid=(ntq,),
        in_specs=[row_spec, row_spec, row_spec,
                  full_spec((EMB, 64)), full_spec((EMB, 64)),
                  full_spec((EMB, 64)), full_spec((EMB, 64)),
                  full_spec((EMB, 64)), full_spec((EMB, 64)),
                  full_spec((1, 64)), full_spec((1, 64)), full_spec((1, 64)),
                  full_spec((64, 64)),
                  lane_spec, lane_spec, full_spec((1, 64))],
        out_specs=[lane_spec] * 5,
        out_shape=[jax.ShapeDtypeStruct((NQ, 64), jnp.int32)]
        + [jax.ShapeDtypeStruct((NQ, 64), f32)] * 4,
    )(val0, q0, qp0, wsox_v, wsox_q, wsoy_v, wsoy_q, waw_v, waw_q,
      bsox, bsoy, baw, smask, refx64, refy64, base64)
    ibase, w00, w01, w10, w11 = outs_a

    idx = jnp.pad(ibase, ((0, NQ_PAD - NQ), (0, 0))).reshape(-1, 128)
    wt = jnp.stack([w00, w01, w10, w11], axis=-1).reshape(NQ, EPQ)
    wt = jnp.pad(wt, ((0, NQ_PAD - NQ), (0, 0))).reshape(-1)

    table = pl.pallas_call(
        _vproj_body,
        grid=(NBQ, ntq),
        in_specs=[pl.BlockSpec((1, TQ, EMB), lambda b, i: (b, i, 0)),
                  pl.BlockSpec((EMB, EMB), lambda b, i: (0, 0)),
                  pl.BlockSpec((1, EMB), lambda b, i: (0, 0))],
        out_specs=pl.BlockSpec((1, NH, TQ, DH), lambda b, i: (b, 0, i, 0)),
        out_shape=jax.ShapeDtypeStruct((NBQ, NH, NQ, DH), f32),
    )(value, W_vp.T, b_vp.reshape(1, EMB))
    vg = table.reshape(NBQ, NH, H0, W0, DH)
    vp = jnp.pad(vg, ((0, 0), (0, 0), (1, 2), (1, 2), (0, 0)))
    t4 = jnp.stack([vp[:, :, 0:HP, 0:WP], vp[:, :, 0:HP, 1:WP + 1],
                    vp[:, :, 1:HP + 1, 0:WP], vp[:, :, 1:HP + 1, 1:WP + 1]],
                   axis=4)
    table = t4.reshape(NBQ * NH * HP * WP, 4 * DH)

    mesh = plsc.VectorSubcoreMesh(core_axis_name="c", subcore_axis_name="s")
    sc_fn = functools.partial(
        pl.kernel,
        mesh=mesh,
        out_type=jax.ShapeDtypeStruct((NQ_PAD, EMB), f32),
        scratch_types=[
            pltpu.VMEM((2, 128), jnp.int32),
            pltpu.VMEM((2, 128), jnp.int32),
            pltpu.VMEM((K + 16,), f32),
            pltpu.VMEM((KR, 4 * DH), f32),
            pltpu.VMEM((KR, 4 * DH), f32),
            pltpu.VMEM((G, EMB), f32),
            pltpu.SemaphoreType.DMA,
            pltpu.SemaphoreType.DMA,
        ],
    )(_sc_gather_kernel)
    msda_pad = sc_fn(table, idx, wt)

    msda = msda_pad[:NQ]
    out = pl.pallas_call(
        _stage_c_body,
        grid=(ntq,),
        in_specs=[row_spec, full_spec((EMB, EMB)), full_spec((1, EMB)),
                  row_spec],
        out_specs=row_spec,
        out_shape=jax.ShapeDtypeStruct((NQ, EMB), f32),
    )(msda, W_op.T, b_op.reshape(1, EMB), q0)
    return out.reshape(1, NQ, EMB)


# speedup vs baseline: 29.7326x; 1.0003x over previous
import functools
import numpy as np
import jax
import jax.numpy as jnp
from jax import lax
from jax.experimental import pallas as pl
from jax.experimental.pallas import tpu as pltpu
from jax.experimental.pallas import tpu_sc as plsc

NQ = 10000
EMB = 256
NH = 8
NP = 4
NBQ = 2
H0 = 100
W0 = 100
DH = EMB // NH
LPH = NBQ * NP * 4
EPQ = NH * LPH

NW = 32
NQ_PAD = 10240
QPW = NQ_PAD // NW
G = 4
CPW = QPW // G
K = G * EPQ
SPQ = NH * NBQ * NP
KR = G * SPQ
HP = H0 + 2
WP = W0 + 2

TQ = 1000


def _stage_a_body(val0_ref, q_ref, qp_ref, wsox_v_ref, wsox_q_ref,
                  wsoy_v_ref, wsoy_q_ref, waw_v_ref, waw_q_ref,
                  bsox_ref, bsoy_ref, baw_ref, smask_ref,
                  refx_ref, refy_ref, base_ref,
                  ib_ref, w00_ref, w01_ref, w10_ref, w11_ref):
    f32 = jnp.float32
    v0 = val0_ref[...]
    qq = q_ref[...] + qp_ref[...]
    sox = (jnp.dot(v0, wsox_v_ref[...], preferred_element_type=f32)
           + jnp.dot(qq, wsox_q_ref[...], preferred_element_type=f32)
           + bsox_ref[...])
    soy = (jnp.dot(v0, wsoy_v_ref[...], preferred_element_type=f32)
           + jnp.dot(qq, wsoy_q_ref[...], preferred_element_type=f32)
           + bsoy_ref[...])
    logits = (jnp.dot(v0, waw_v_ref[...], preferred_element_type=f32)
              + jnp.dot(qq, waw_q_ref[...], preferred_element_type=f32)
              + baw_ref[...])
    e = jnp.exp(logits)
    denom = jnp.dot(e, smask_ref[...], preferred_element_type=f32)
    aw = e / denom

    px = refx_ref[...] * W0 + sox - 0.5
    py = refy_ref[...] * H0 + soy - 0.5
    x0f = jnp.floor(px)
    y0f = jnp.floor(py)
    fx = px - x0f
    fy = py - y0f
    x1f = x0f + 1.0
    y1f = y0f + 1.0
    vx0 = (x0f >= 0.0) & (x0f <= W0 - 1.0)
    vx1 = (x1f >= 0.0) & (x1f <= W0 - 1.0)
    vy0 = (y0f >= 0.0) & (y0f <= H0 - 1.0)
    vy1 = (y1f >= 0.0) & (y1f <= H0 - 1.0)
    xbi = (jnp.clip(x0f, -1.0, float(W0)) + 1.0).astype(jnp.int32)
    ybi = (jnp.clip(y0f, -1.0, float(H0)) + 1.0).astype(jnp.int32)

    base = base_ref[...]
    ib_ref[...] = base + ybi * WP + xbi
    half_aw = 0.5 * aw
    wx0 = 1.0 - fx
    wy0 = 1.0 - fy
    z = jnp.zeros_like(aw)
    w00_ref[...] = jnp.where(vx0 & vy0, half_aw * wx0 * wy0, z)
    w01_ref[...] = jnp.where(vx1 & vy0, half_aw * fx * wy0, z)
    w10_ref[...] = jnp.where(vx0 & vy1, half_aw * wx0 * fy, z)
    w11_ref[...] = jnp.where(vx1 & vy1, half_aw * fx * fy, z)


def _vproj_body(val_ref, wvp_ref, bvp_ref, out_ref):
    vt = (jnp.dot(val_ref[0], wvp_ref[...], preferred_element_type=jnp.float32)
          + bvp_ref[...])
    for h in range(NH):
        out_ref[0, h] = vt[:, h * DH:(h + 1) * DH]


def _stage_c_body(x_ref, wop_ref, bop_ref, q_ref, out_ref):
    out_ref[...] = (jnp.dot(x_ref[...], wop_ref[...],
                            preferred_element_type=jnp.float32)
                    + bop_ref[...] + q_ref[...])


def _sc_gather_kernel(table_hbm, idx_hbm, wt_hbm, out_hbm,
                      idx_a, idx_b, wt_v, rows_a, rows_b, out_v,
                      sem_a, sem_b):
    wid = lax.axis_index("s") * 2 + lax.axis_index("c")
    ibase = wid * (CPW * 2)
    ebase = wid * (QPW * EPQ)
    qbase = wid * QPW
    c4v = [jnp.full((16, 1), i, jnp.int32) for i in range(4)]
    gd = lax.GatherDimensionNumbers(offset_dims=(), collapsed_slice_dims=(0,),
                                    start_index_map=(0,))

    def lane_splat(vec, idx):
        return lax.gather(vec, idx, gd, slice_sizes=(1,),
                          mode=lax.GatherScatterMode.PROMISE_IN_BOUNDS)

    def start_gathers(c, idx_v, rows_v, sem):
        pltpu.sync_copy(idx_hbm.at[pl.ds(ibase + c * 2, 2)], idx_v)
        for i in range(2):
            pltpu.async_copy(table_hbm.at[idx_v.at[i]],
                             rows_v.at[pl.ds(i * 128, 128)], sem)

    def drain_gathers(idx_v, rows_v, sem):
        for i in range(2):
            pltpu.make_async_copy(table_hbm.at[idx_v.at[i]],
                                  rows_v.at[pl.ds(i * 128, 128)], sem).wait()

    def compute_chunk(c, rows_v):
        pltpu.sync_copy(wt_hbm.at[pl.ds(ebase + c * K, K)],
                        wt_v.at[pl.ds(0, K)])
        for g in range(G):
            for h in range(NH):
                bw = g * EPQ + h * (NBQ * NP * 4)
                br = g * SPQ + h * (NBQ * NP)

                def sbody(s, acc):
                    a0, a1 = acc
                    w4 = wt_v[pl.ds(bw + s * 4, 16)]
                    r = br + s
                    for c4 in range(4):
                        wc = lane_splat(w4, c4v[c4])
                        a0 = a0 + wc * rows_v[r, pl.ds(c4 * DH, 16)]
                        a1 = a1 + wc * rows_v[r, pl.ds(c4 * DH + 16, 16)]
                    return (a0, a1)

                acc0, acc1 = lax.fori_loop(
                    0, NBQ * NP, sbody,
                    (jnp.zeros((16,), jnp.float32),
                     jnp.zeros((16,), jnp.float32)))
                out_v[g, pl.ds(h * DH, 16)] = acc0
                out_v[g, pl.ds(h * DH + 16, 16)] = acc1
        pltpu.sync_copy(out_v, out_hbm.at[pl.ds(qbase + c * G, G)])

    start_gathers(0, idx_a, rows_a, sem_a)

    def pair_body(t, carry):
        c0 = 2 * t
        start_gathers(c0 + 1, idx_b, rows_b, sem_b)
        drain_gathers(idx_a, rows_a, sem_a)
        compute_chunk(c0, rows_a)
        start_gathers(jnp.minimum(c0 + 2, CPW - 1), idx_a, rows_a, sem_a)
        drain_gathers(idx_b, rows_b, sem_b)
        compute_chunk(c0 + 1, rows_b)
        return carry

    lax.fori_loop(0, CPW // 2, pair_body, 0)
    drain_gathers(idx_a, rows_a, sem_a)


def kernel(query, key, value, query_pos, reference_points, spatial_shapes,
           level_start_index, im2col_step, W_so, b_so, W_aw, b_aw,
           W_vp, b_vp, W_op, b_op):
    f32 = jnp.float32
    q0 = query[0]
    qp0 = query_pos[0]
    val0 = value[0]

    wsox = W_so[0::2].T
    wsoy = W_so[1::2].T
    waw = W_aw.T
    wsox_v, wsox_q = wsox[:EMB], wsox[EMB:]
    wsoy_v, wsoy_q = wsoy[:EMB], wsoy[EMB:]
    waw_v, waw_q = waw[:EMB], waw[EMB:]
    bsox = b_so[0::2].reshape(1, 64)
    bsoy = b_so[1::2].reshape(1, 64)
    baw = b_aw.reshape(1, 64)

    lidx = np.arange(64)
    smask = jnp.asarray((lidx[:, None] // NP) == (lidx[None, :] // NP), f32)
    hh = lidx // (NBQ * NP)
    bb = (lidx // NP) % NBQ
    base64 = jnp.asarray(((bb * NH + hh) * (HP * WP))[None, :], jnp.int32)

    refx = reference_points[:, :, 0, 0]
    refy = reference_points[:, :, 0, 1]
    refx64 = jnp.tile(jnp.repeat(refx.T, NP, axis=1), (1, NH))
    refy64 = jnp.tile(jnp.repeat(refy.T, NP, axis=1), (1, NH))

    ntq = NQ // TQ
    row_spec = pl.BlockSpec((TQ, EMB), lambda i: (i, 0))
    lane_spec = pl.BlockSpec((TQ, 64), lambda i: (i, 0))
    full_spec = lambda s: pl.BlockSpec(s, lambda i: tuple(0 for _ in s))
    outs_a = pl.pallas_call(
        _stage_a_body,
        grid=(ntq,),
        in_specs=[row_spec, row_spec, row_spec,
                  full_spec((EMB, 64)), full_spec((EMB, 64)),
                  full_spec((EMB, 64)), full_spec((EMB, 64)),
                  full_spec((EMB, 64)), full_spec((EMB, 64)),
                  full_spec((1, 64)), full_spec((1, 64)), full_spec((1, 64)),
                  full_spec((64, 64)),
                  lane_spec, lane_spec, full_spec((1, 64))],
        out_specs=[lane_spec] * 5,
        out_shape=[jax.ShapeDtypeStruct((NQ, 64), jnp.int32)]
        + [jax.ShapeDtypeStruct((NQ, 64), f32)] * 4,
    )(val0, q0, qp0, wsox_v, wsox_q, wsoy_v, wsoy_q, waw_v, waw_q,
      bsox, bsoy, baw, smask, refx64, refy64, base64)
    ibase, w00, w01, w10, w11 = outs_a

    idx = jnp.pad(ibase, ((0, NQ_PAD - NQ), (0, 0))).reshape(-1, 128)
    wt = jnp.stack([w00, w01, w10, w11], axis=-1).reshape(NQ, EPQ)
    wt = jnp.pad(wt, ((0, NQ_PAD - NQ), (0, 0))).reshape(-1)

    table = pl.pallas_call(
        _vproj_body,
        grid=(NBQ, ntq),
        in_specs=[pl.BlockSpec((1, TQ, EMB), lambda b, i: (b, i, 0)),
                  pl.BlockSpec((EMB, EMB), lambda b, i: (0, 0)),
                  pl.BlockSpec((1, EMB), lambda b, i: (0, 0))],
        out_specs=pl.BlockSpec((1, NH, TQ, DH), lambda b, i: (b, 0, i, 0)),
        out_shape=jax.ShapeDtypeStruct((NBQ, NH, NQ, DH), f32),
    )(value, W_vp.T, b_vp.reshape(1, EMB))
    vg = table.reshape(NBQ, NH, H0, W0, DH)
    vp = jnp.pad(vg, ((0, 0), (0, 0), (1, 2), (1, 2), (0, 0)))
    t4 = jnp.stack([vp[:, :, 0:HP, 0:WP], vp[:, :, 0:HP, 1:WP + 1],
                    vp[:, :, 1:HP + 1, 0:WP], vp[:, :, 1:HP + 1, 1:WP + 1]],
                   axis=4)
    table = t4.reshape(NBQ * NH * HP * WP, 4 * DH)

    mesh = plsc.VectorSubcoreMesh(core_axis_name="c", subcore_axis_name="s")
    sc_fn = functools.partial(
        pl.kernel,
        mesh=mesh,
        out_type=jax.ShapeDtypeStruct((NQ_PAD, EMB), f32),
        scratch_types=[
            pltpu.VMEM((2, 128), jnp.int32),
            pltpu.VMEM((2, 128), jnp.int32),
            pltpu.VMEM((K + 16,), f32),
            pltpu.VMEM((KR, 4 * DH), f32),
            pltpu.VMEM((KR, 4 * DH), f32),
            pltpu.VMEM((G, EMB), f32),
            pltpu.SemaphoreType.DMA,
            pltpu.SemaphoreType.DMA,
        ],
    )(_sc_gather_kernel)
    msda_pad = sc_fn(table, idx, wt)

    msda = msda_pad[:NQ]
    out = pl.pallas_call(
        _stage_c_body,
        grid=(ntq,),
        in_specs=[row_spec, full_spec((EMB, EMB)), full_spec((1, EMB)),
                  row_spec],
        out_specs=row_spec,
        out_shape=jax.ShapeDtypeStruct((NQ, EMB), f32),
    )(msda, W_op.T, b_op.reshape(1, EMB), q0)
    return out.reshape(1, NQ, EMB)


# speedup vs baseline: 30.0468x; 1.0106x over previous
import functools
import numpy as np
import jax
import jax.numpy as jnp
from jax import lax
from jax.experimental import pallas as pl
from jax.experimental.pallas import tpu as pltpu
from jax.experimental.pallas import tpu_sc as plsc

NQ = 10000
EMB = 256
NH = 8
NP = 4
NBQ = 2
H0 = 100
W0 = 100
DH = EMB // NH
LPH = NBQ * NP * 4
EPQ = NH * LPH

NW = 32
NQ_PAD = 10240
QPW = NQ_PAD // NW
G = 4
CPW = QPW // G
K = G * EPQ
SPQ = NH * NBQ * NP
KR = G * SPQ
HP = H0 + 2
WP = W0 + 2

TQ = 1000


def _stage_a_body(val0_ref, q_ref, qp_ref, wsox_v_ref, wsox_q_ref,
                  wsoy_v_ref, wsoy_q_ref, waw_v_ref, waw_q_ref,
                  bsox_ref, bsoy_ref, baw_ref, smask_ref,
                  refx_ref, refy_ref, base_ref,
                  ib_ref, w00_ref, w01_ref, w10_ref, w11_ref):
    f32 = jnp.float32
    v0 = val0_ref[...]
    qq = q_ref[...] + qp_ref[...]
    sox = (jnp.dot(v0, wsox_v_ref[...], preferred_element_type=f32)
           + jnp.dot(qq, wsox_q_ref[...], preferred_element_type=f32)
           + bsox_ref[...])
    soy = (jnp.dot(v0, wsoy_v_ref[...], preferred_element_type=f32)
           + jnp.dot(qq, wsoy_q_ref[...], preferred_element_type=f32)
           + bsoy_ref[...])
    logits = (jnp.dot(v0, waw_v_ref[...], preferred_element_type=f32)
              + jnp.dot(qq, waw_q_ref[...], preferred_element_type=f32)
              + baw_ref[...])
    e = jnp.exp(logits)
    denom = jnp.dot(e, smask_ref[...], preferred_element_type=f32)
    aw = e / denom

    px = refx_ref[...] * W0 + sox - 0.5
    py = refy_ref[...] * H0 + soy - 0.5
    x0f = jnp.floor(px)
    y0f = jnp.floor(py)
    fx = px - x0f
    fy = py - y0f
    x1f = x0f + 1.0
    y1f = y0f + 1.0
    vx0 = (x0f >= 0.0) & (x0f <= W0 - 1.0)
    vx1 = (x1f >= 0.0) & (x1f <= W0 - 1.0)
    vy0 = (y0f >= 0.0) & (y0f <= H0 - 1.0)
    vy1 = (y1f >= 0.0) & (y1f <= H0 - 1.0)
    xbi = (jnp.clip(x0f, -1.0, float(W0)) + 1.0).astype(jnp.int32)
    ybi = (jnp.clip(y0f, -1.0, float(H0)) + 1.0).astype(jnp.int32)

    base = base_ref[...]
    ib_ref[...] = base + ybi * WP + xbi
    half_aw = 0.5 * aw
    wx0 = 1.0 - fx
    wy0 = 1.0 - fy
    z = jnp.zeros_like(aw)
    w00_ref[...] = jnp.where(vx0 & vy0, half_aw * wx0 * wy0, z)
    w01_ref[...] = jnp.where(vx1 & vy0, half_aw * fx * wy0, z)
    w10_ref[...] = jnp.where(vx0 & vy1, half_aw * wx0 * fy, z)
    w11_ref[...] = jnp.where(vx1 & vy1, half_aw * fx * fy, z)


def _vproj_body(val_ref, wvp_ref, bvp_ref, out_ref):
    vt = (jnp.dot(val_ref[0], wvp_ref[...], preferred_element_type=jnp.float32)
          + bvp_ref[...])
    for h in range(NH):
        out_ref[0, h] = vt[:, h * DH:(h + 1) * DH]


def _stage_c_body(x_ref, wop_ref, bop_ref, q_ref, out_ref):
    out_ref[...] = (jnp.dot(x_ref[...], wop_ref[...],
                            preferred_element_type=jnp.float32)
                    + bop_ref[...] + q_ref[...])


def _sc_gather_kernel(table_hbm, idx_hbm, wt_hbm, out_hbm,
                      idx_a, idx_b, wt_a, wt_b, rows_a, rows_b, out_v,
                      sem_a, sem_b):
    wid = lax.axis_index("s") * 2 + lax.axis_index("c")
    ibase = wid * (CPW * 2)
    ebase = wid * (QPW * EPQ)
    qbase = wid * QPW
    c4v = [jnp.full((16, 1), i, jnp.int32) for i in range(4)]
    gd = lax.GatherDimensionNumbers(offset_dims=(), collapsed_slice_dims=(0,),
                                    start_index_map=(0,))

    def lane_splat(vec, idx):
        return lax.gather(vec, idx, gd, slice_sizes=(1,),
                          mode=lax.GatherScatterMode.PROMISE_IN_BOUNDS)

    def start_gathers(c, idx_v, rows_v, wt_v, sem):
        pltpu.sync_copy(idx_hbm.at[pl.ds(ibase + c * 2, 2)], idx_v)
        for i in range(2):
            pltpu.async_copy(table_hbm.at[idx_v.at[i]],
                             rows_v.at[pl.ds(i * 128, 128)], sem)
        pltpu.async_copy(wt_hbm.at[pl.ds(ebase + c * K, K)],
                         wt_v.at[pl.ds(0, K)], sem)

    def drain_gathers(idx_v, rows_v, wt_v, sem):
        for i in range(2):
            pltpu.make_async_copy(table_hbm.at[idx_v.at[i]],
                                  rows_v.at[pl.ds(i * 128, 128)], sem).wait()
        pltpu.make_async_copy(wt_hbm.at[pl.ds(0, K)],
                              wt_v.at[pl.ds(0, K)], sem).wait()

    def compute_chunk(c, rows_v, wt_v):
        for g in range(G):
            for h in range(NH):
                bw = g * EPQ + h * (NBQ * NP * 4)
                br = g * SPQ + h * (NBQ * NP)

                def sbody(s, acc):
                    a0, a1 = acc
                    w4 = wt_v[pl.ds(bw + s * 4, 16)]
                    r = br + s
                    for c4 in range(4):
                        wc = lane_splat(w4, c4v[c4])
                        a0 = a0 + wc * rows_v[r, pl.ds(c4 * DH, 16)]
                        a1 = a1 + wc * rows_v[r, pl.ds(c4 * DH + 16, 16)]
                    return (a0, a1)

                acc0, acc1 = lax.fori_loop(
                    0, NBQ * NP, sbody,
                    (jnp.zeros((16,), jnp.float32),
                     jnp.zeros((16,), jnp.float32)))
                out_v[g, pl.ds(h * DH, 16)] = acc0
                out_v[g, pl.ds(h * DH + 16, 16)] = acc1
        pltpu.sync_copy(out_v, out_hbm.at[pl.ds(qbase + c * G, G)])

    start_gathers(0, idx_a, rows_a, wt_a, sem_a)

    def pair_body(t, carry):
        c0 = 2 * t
        start_gathers(c0 + 1, idx_b, rows_b, wt_b, sem_b)
        drain_gathers(idx_a, rows_a, wt_a, sem_a)
        compute_chunk(c0, rows_a, wt_a)
        start_gathers(jnp.minimum(c0 + 2, CPW - 1), idx_a, rows_a, wt_a,
                      sem_a)
        drain_gathers(idx_b, rows_b, wt_b, sem_b)
        compute_chunk(c0 + 1, rows_b, wt_b)
        return carry

    lax.fori_loop(0, CPW // 2, pair_body, 0)
    drain_gathers(idx_a, rows_a, wt_a, sem_a)


def kernel(query, key, value, query_pos, reference_points, spatial_shapes,
           level_start_index, im2col_step, W_so, b_so, W_aw, b_aw,
           W_vp, b_vp, W_op, b_op):
    f32 = jnp.float32
    q0 = query[0]
    qp0 = query_pos[0]
    val0 = value[0]

    wsox = W_so[0::2].T
    wsoy = W_so[1::2].T
    waw = W_aw.T
    wsox_v, wsox_q = wsox[:EMB], wsox[EMB:]
    wsoy_v, wsoy_q = wsoy[:EMB], wsoy[EMB:]
    waw_v, waw_q = waw[:EMB], waw[EMB:]
    bsox = b_so[0::2].reshape(1, 64)
    bsoy = b_so[1::2].reshape(1, 64)
    baw = b_aw.reshape(1, 64)

    lidx = np.arange(64)
    smask = jnp.asarray((lidx[:, None] // NP) == (lidx[None, :] // NP), f32)
    hh = lidx // (NBQ * NP)
    bb = (lidx // NP) % NBQ
    base64 = jnp.asarray(((bb * NH + hh) * (HP * WP))[None, :], jnp.int32)

    refx = reference_points[:, :, 0, 0]
    refy = reference_points[:, :, 0, 1]
    refx64 = jnp.tile(jnp.repeat(refx.T, NP, axis=1), (1, NH))
    refy64 = jnp.tile(jnp.repeat(refy.T, NP, axis=1), (1, NH))

    ntq = NQ // TQ
    row_spec = pl.BlockSpec((TQ, EMB), lambda i: (i, 0))
    lane_spec = pl.BlockSpec((TQ, 64), lambda i: (i, 0))
    full_spec = lambda s: pl.BlockSpec(s, lambda i: tuple(0 for _ in s))
    outs_a = pl.pallas_call(
        _stage_a_body,
        grid=(ntq,),
        in_specs=[row_spec, row_spec, row_spec,
                  full_spec((EMB, 64)), full_spec((EMB, 64)),
                  full_spec((EMB, 64)), full_spec((EMB, 64)),
                  full_spec((EMB, 64)), full_spec((EMB, 64)),
                  full_spec((1, 64)), full_spec((1, 64)), full_spec((1, 64)),
                  full_spec((64, 64)),
                  lane_spec, lane_spec, full_spec((1, 64))],
        out_specs=[lane_spec] * 5,
        out_shape=[jax.ShapeDtypeStruct((NQ, 64), jnp.int32)]
        + [jax.ShapeDtypeStruct((NQ, 64), f32)] * 4,
    )(val0, q0, qp0, wsox_v, wsox_q, wsoy_v, wsoy_q, waw_v, waw_q,
      bsox, bsoy, baw, smask, refx64, refy64, base64)
    ibase, w00, w01, w10, w11 = outs_a

    idx = jnp.pad(ibase, ((0, NQ_PAD - NQ), (0, 0))).reshape(-1, 128)
    wt = jnp.stack([w00, w01, w10, w11], axis=-1).reshape(NQ, EPQ)
    wt = jnp.pad(wt, ((0, NQ_PAD - NQ), (0, 0))).reshape(-1)

    table = pl.pallas_call(
        _vproj_body,
        grid=(NBQ, ntq),
        in_specs=[pl.BlockSpec((1, TQ, EMB), lambda b, i: (b, i, 0)),
                  pl.BlockSpec((EMB, EMB), lambda b, i: (0, 0)),
                  pl.BlockSpec((1, EMB), lambda b, i: (0, 0))],
        out_specs=pl.BlockSpec((1, NH, TQ, DH), lambda b, i: (b, 0, i, 0)),
        out_shape=jax.ShapeDtypeStruct((NBQ, NH, NQ, DH), f32),
    )(value, W_vp.T, b_vp.reshape(1, EMB))
    vg = table.reshape(NBQ, NH, H0, W0, DH)
    vp = jnp.pad(vg, ((0, 0), (0, 0), (1, 2), (1, 2), (0, 0)))
    t4 = jnp.stack([vp[:, :, 0:HP, 0:WP], vp[:, :, 0:HP, 1:WP + 1],
                    vp[:, :, 1:HP + 1, 0:WP], vp[:, :, 1:HP + 1, 1:WP + 1]],
                   axis=4)
    table = t4.reshape(NBQ * NH * HP * WP, 4 * DH)

    mesh = plsc.VectorSubcoreMesh(core_axis_name="c", subcore_axis_name="s")
    sc_fn = functools.partial(
        pl.kernel,
        mesh=mesh,
        out_type=jax.ShapeDtypeStruct((NQ_PAD, EMB), f32),
        scratch_types=[
            pltpu.VMEM((2, 128), jnp.int32),
            pltpu.VMEM((2, 128), jnp.int32),
            pltpu.VMEM((K + 16,), f32),
            pltpu.VMEM((K + 16,), f32),
            pltpu.VMEM((KR, 4 * DH), f32),
            pltpu.VMEM((KR, 4 * DH), f32),
            pltpu.VMEM((G, EMB), f32),
            pltpu.SemaphoreType.DMA,
            pltpu.SemaphoreType.DMA,
        ],
    )(_sc_gather_kernel)
    msda_pad = sc_fn(table, idx, wt)

    msda = msda_pad[:NQ]
    out = pl.pallas_call(
        _stage_c_body,
        grid=(ntq,),
        in_specs=[row_spec, full_spec((EMB, EMB)), full_spec((1, EMB)),
                  row_spec],
        out_specs=row_spec,
        out_shape=jax.ShapeDtypeStruct((NQ, EMB), f32),
    )(msda, W_op.T, b_op.reshape(1, EMB), q0)
    return out.reshape(1, NQ, EMB)
